# Initial kernel scaffold; baseline (speedup 1.0000x reference)
import jax
import jax.numpy as jnp
from jax.experimental import pallas as pl


def kernel(n_feat, e_feat, W_M, b_M, W_U, b_U, gamma, beta, W_mix, b_mix, edge_index):
    raise NotImplementedError("write your pallas kernel here")


# speedup vs baseline: 1.0521x; 1.0521x over previous
import functools
import jax
import jax.numpy as jnp
from jax.experimental import pallas as pl
from jax.experimental.pallas import tpu as pltpu

N = 10000
E = 320000
D = 128
DELTA = 3.5


def _mm_kernel(x_ref, w_ref, b_ref, o_ref):
    o_ref[...] = jnp.dot(x_ref[...], w_ref[...],
                         preferred_element_type=jnp.float32) + b_ref[...]


def _matmul(x, w, b, blk):
    M, K = x.shape
    Ko, F = w.shape
    grid = (M // blk,)
    return pl.pallas_call(
        _mm_kernel,
        grid=grid,
        in_specs=[pl.BlockSpec((blk, K), lambda i: (i, 0)),
                  pl.BlockSpec((K, F), lambda i: (0, 0)),
                  pl.BlockSpec((1, F), lambda i: (0, 0))],
        out_specs=pl.BlockSpec((blk, F), lambda i: (i, 0)),
        out_shape=jax.ShapeDtypeStruct((M, F), jnp.float32),
    )(x, w, b.reshape(1, F))


def kernel(n_feat, e_feat, W_M, b_M, W_U, b_U, gamma, beta, W_mix, b_mix, edge_index):
    src = edge_index[0]
    dst = edge_index[1]
    W_AB = jnp.concatenate([W_M[:D], W_M[D:2*D]], axis=1)
    b_AB = jnp.concatenate([jnp.zeros_like(b_M), b_M])
    AB = _matmul(n_feat, W_AB, b_AB, 400)
    A, B = AB[:, :D], AB[:, D:]
    C = _matmul(e_feat, W_M[2*D:], jnp.zeros_like(b_M), 512)

    m = jnp.take(A, src, axis=0) + C
    ones = jnp.ones((E,), jnp.float32)
    deg = jax.ops.segment_sum(ones, dst, num_segments=N)
    Sm = jax.ops.segment_sum(m, dst, num_segments=N)
    Sm2 = jax.ops.segment_sum(m * m, dst, num_segments=N)
    Mx = jax.ops.segment_max(m, dst, num_segments=N)
    Mn = jax.ops.segment_min(m, dst, num_segments=N)

    has = (deg > 0)[:, None]
    safe = jnp.where(deg > 0, deg, 1.0)[:, None]
    s_full = Sm + deg[:, None] * B
    ssq_full = Sm2 + 2 * B * Sm + deg[:, None] * B * B
    mean = s_full / safe
    mean_sq = ssq_full / safe
    var = jax.nn.relu(mean_sq - mean * mean)
    std = jnp.sqrt(var + 1e-30)
    mx = jnp.where(has, Mx + B, 0.0)
    mn = jnp.where(has, Mn + B, 0.0)
    h = jnp.concatenate([mean, mx, mn, std], axis=1)
    logd = jnp.log(deg + 1.0)[:, None]
    amp = logd / DELTA
    att = jnp.where(logd > 0, DELTA / jnp.where(logd > 0, logd, 1.0), 0.0)
    h_scaled = jnp.concatenate([h, h * amp, h * att], axis=1)
    hcat = jnp.concatenate([n_feat, h_scaled], axis=-1)
    h_post = _matmul(hcat, W_U, b_U, 400) * jnp.sqrt(1.0 / N).astype(jnp.float32)
    mu = jnp.mean(h_post, axis=0)
    v = jnp.mean((h_post - mu) ** 2, axis=0)
    h_bn = (h_post - mu) / jnp.sqrt(v + 1e-5) * gamma + beta
    h_mix = _matmul(h_bn, W_mix, b_mix, 400)
    h_mix = jnp.where(h_mix >= 0, h_mix, 0.01 * h_mix)
    return jax.nn.relu(h_mix + n_feat)
